# Initial kernel scaffold; baseline (speedup 1.0000x reference)
import jax
import jax.numpy as jnp
from jax.experimental import pallas as pl


def kernel(x, batch, W_fx, b_fx, W_x, b_x, W_slice, b_slice, Wq, Wk, Wv, W_out, b_out, g_temp):
    raise NotImplementedError("write your pallas kernel here")


# speedup vs baseline: 31.4331x; 31.4331x over previous
import functools

import jax
import jax.numpy as jnp
from jax.experimental import pallas as pl
from jax.experimental.pallas import tpu as pltpu

_B = 16


def _slice_softmax(xb, wx, bx, wsl, bsl, invg, gsum):
    xm = jnp.dot(xb, wx, preferred_element_type=jnp.float32) + bx
    logits = (jnp.dot(xm, wsl, preferred_element_type=jnp.float32) + bsl) * invg
    m = jnp.max(logits, axis=-1, keepdims=True)
    e = jnp.exp(logits - m)
    denom = jnp.dot(e, gsum, preferred_element_type=jnp.float32)
    return e / denom


def _onehot512(bcol, bs):
    bid = (jax.lax.broadcasted_iota(jnp.int32, (1, bs), 1) // (bs // _B))
    return (bcol == bid.astype(jnp.float32)).astype(jnp.float32)


def _accum_kernel(heads, slices, xb_ref, bcol_ref, wfx_ref, bfx_ref, wx_ref,
                  bx_ref, wsl_ref, bsl_ref, invg_ref, gsum_ref, acc_ref):
    i = pl.program_id(0)

    @pl.when(i == 0)
    def _init():
        acc_ref[...] = jnp.zeros_like(acc_ref)

    xb = xb_ref[...]
    fx = jnp.dot(xb, wfx_ref[...], preferred_element_type=jnp.float32) + bfx_ref[...]
    sw = _slice_softmax(xb, wx_ref[...], bx_ref[...], wsl_ref[...],
                        bsl_ref[...], invg_ref[...], gsum_ref[...])
    bs = _B * slices
    onehot = _onehot512(bcol_ref[...], bs)
    ones = jnp.ones((xb.shape[0], 1), jnp.float32)
    for h in range(heads):
        sw_h = sw[:, h * slices:(h + 1) * slices]
        comb = onehot * jnp.concatenate([sw_h] * _B, axis=1)
        fxa = jnp.concatenate([fx[:, h * slices:(h + 1) * slices], ones], axis=1)
        res = jax.lax.dot_general(comb, fxa, (((0,), (0,)), ((), ())),
                                  preferred_element_type=jnp.float32)
        acc_ref[h] = acc_ref[h] + res


def _out_kernel(heads, slices, dh, xb_ref, bcol_ref, wx_ref, bx_ref, wsl_ref,
                bsl_ref, invg_ref, gsum_ref, acc_ref, wq_ref, wk_ref, wv_ref,
                wout_ref, bout_ref, out_ref, att_ref):
    i = pl.program_id(0)
    bs = _B * slices
    scale = float(dh) ** -0.5

    @pl.when(i == 0)
    def _attend():
        r = jax.lax.broadcasted_iota(jnp.int32, (bs, bs), 0) // slices
        c = jax.lax.broadcasted_iota(jnp.int32, (bs, bs), 1) // slices
        same_graph = r == c
        for h in range(heads):
            a = acc_ref[h]
            tok = a[:, :dh] / (a[:, dh:dh + 1] + 1e-5)
            q = jnp.dot(tok, wq_ref[...], preferred_element_type=jnp.float32)
            k = jnp.dot(tok, wk_ref[...], preferred_element_type=jnp.float32)
            v = jnp.dot(tok, wv_ref[...], preferred_element_type=jnp.float32)
            dots = jax.lax.dot_general(q, k, (((1,), (1,)), ((), ())),
                                       preferred_element_type=jnp.float32)
            dots = jnp.where(same_graph, dots * scale, -1e30)
            mx = jnp.max(dots, axis=-1, keepdims=True)
            e = jnp.exp(dots - mx)
            attn = e / jnp.sum(e, axis=-1, keepdims=True)
            att_ref[h] = jnp.dot(attn, v, preferred_element_type=jnp.float32)

    xb = xb_ref[...]
    sw = _slice_softmax(xb, wx_ref[...], bx_ref[...], wsl_ref[...],
                        bsl_ref[...], invg_ref[...], gsum_ref[...])
    onehot = _onehot512(bcol_ref[...], bs)
    outs = []
    for h in range(heads):
        sw_h = sw[:, h * slices:(h + 1) * slices]
        comb = onehot * jnp.concatenate([sw_h] * _B, axis=1)
        outs.append(jnp.dot(comb, att_ref[h], preferred_element_type=jnp.float32))
    mid = jnp.concatenate(outs, axis=1)
    out_ref[...] = (jnp.dot(mid, wout_ref[...], preferred_element_type=jnp.float32)
                    + bout_ref[...])


def kernel(x, batch, W_fx, b_fx, W_x, b_x, W_slice, b_slice, Wq, Wk, Wv,
           W_out, b_out, g_temp):
    n, dim = x.shape
    heads = g_temp.shape[1]
    inner = W_fx.shape[0]
    dh = inner // heads
    slices = W_slice.shape[0]
    bs = _B * slices

    nb = 2000
    grid = (n // nb,)

    bcol = batch.astype(jnp.float32).reshape(n, 1)
    wfx_t = W_fx.T
    wx_t = W_x.T
    eye_h = jnp.eye(heads, dtype=jnp.float32)
    wsl_t = jnp.einsum('hk,sd->hdks', eye_h, W_slice).reshape(inner, heads * slices)
    bsl = jnp.tile(b_slice, heads).reshape(1, heads * slices)
    bfx = b_fx.reshape(1, inner)
    bx = b_x.reshape(1, inner)
    invg = jnp.repeat(1.0 / g_temp.reshape(heads), slices).reshape(1, heads * slices)
    gs_i = jnp.arange(heads * slices) // slices
    gsum = (gs_i[:, None] == gs_i[None, :]).astype(jnp.float32)
    bout = b_out.reshape(1, dim)

    full = lambda r, c: pl.BlockSpec((r, c), lambda i: (0, 0))

    acc = pl.pallas_call(
        functools.partial(_accum_kernel, heads, slices),
        grid=grid,
        in_specs=[
            pl.BlockSpec((nb, dim), lambda i: (i, 0)),
            pl.BlockSpec((nb, 1), lambda i: (i, 0)),
            full(dim, inner), full(1, inner), full(dim, inner), full(1, inner),
            full(inner, heads * slices), full(1, heads * slices),
            full(1, heads * slices), full(heads * slices, heads * slices),
        ],
        out_specs=pl.BlockSpec((heads, bs, dh + 1), lambda i: (0, 0, 0)),
        out_shape=jax.ShapeDtypeStruct((heads, bs, dh + 1), jnp.float32),
    )(x, bcol, wfx_t, bfx, wx_t, bx, wsl_t, bsl, invg, gsum)

    out = pl.pallas_call(
        functools.partial(_out_kernel, heads, slices, dh),
        grid=grid,
        in_specs=[
            pl.BlockSpec((nb, dim), lambda i: (i, 0)),
            pl.BlockSpec((nb, 1), lambda i: (i, 0)),
            full(dim, inner), full(1, inner),
            full(inner, heads * slices), full(1, heads * slices),
            full(1, heads * slices), full(heads * slices, heads * slices),
            pl.BlockSpec((heads, bs, dh + 1), lambda i: (0, 0, 0)),
            full(dh, dh), full(dh, dh), full(dh, dh),
            full(inner, dim), full(1, dim),
        ],
        out_specs=pl.BlockSpec((nb, dim), lambda i: (i, 0)),
        out_shape=jax.ShapeDtypeStruct((n, dim), jnp.float32),
        scratch_shapes=[pltpu.VMEM((heads, bs, dh), jnp.float32)],
    )(x, bcol, wx_t, bx, wsl_t, bsl, invg, gsum, acc,
      Wq.T, Wk.T, Wv.T, W_out.T, bout)
    return out


# speedup vs baseline: 35.0727x; 1.1158x over previous
import functools

import jax
import jax.numpy as jnp
from jax.experimental import pallas as pl
from jax.experimental.pallas import tpu as pltpu

_B = 16


def _fused_kernel(heads, slices, dh, nb, xb_ref, bcol_ref, wfx_ref, bfx_ref,
                  wx_ref, bx_ref, wsl_ref, bsl_ref, invg_ref, gsum_ref,
                  wq_ref, wk_ref, wv_ref, wout_ref, bout_ref,
                  out_ref, sw_ref, acc_ref, att2_ref):
    p = pl.program_id(0)
    i = pl.program_id(1)
    bs = _B * slices
    scale = float(dh) ** -0.5

    def onehot512(bcol):
        bid = jax.lax.broadcasted_iota(jnp.int32, (1, bs), 1) // slices
        return (bcol == bid.astype(jnp.float32)).astype(jnp.float32)

    @pl.when(p == 0)
    def _pool():
        @pl.when(i == 0)
        def _init():
            acc_ref[...] = jnp.zeros_like(acc_ref)

        xb = xb_ref[...]
        fx = jnp.dot(xb, wfx_ref[...], preferred_element_type=jnp.float32) + bfx_ref[...]
        xm = jnp.dot(xb, wx_ref[...], preferred_element_type=jnp.float32) + bx_ref[...]
        logits = (jnp.dot(xm, wsl_ref[...], preferred_element_type=jnp.float32)
                  + bsl_ref[...]) * invg_ref[...]
        m = jnp.max(logits, axis=-1, keepdims=True)
        e = jnp.exp(logits - m)
        denom = jnp.dot(e, gsum_ref[...], preferred_element_type=jnp.float32)
        sw = e / denom
        sw_ref[pl.ds(i * nb, nb), :] = sw
        onehot = onehot512(bcol_ref[...])
        ones = jnp.ones((nb, 1), jnp.float32)
        for h in range(heads):
            sw_h = sw[:, h * slices:(h + 1) * slices]
            comb = onehot * jnp.concatenate([sw_h] * _B, axis=1)
            fxa = jnp.concatenate([fx[:, h * slices:(h + 1) * slices], ones],
                                  axis=1)
            res = jax.lax.dot_general(comb, fxa, (((0,), (0,)), ((), ())),
                                      preferred_element_type=jnp.float32)
            acc_ref[h] = acc_ref[h] + res

    @pl.when((p == 1) & (i == 0))
    def _attend():
        r = jax.lax.broadcasted_iota(jnp.int32, (bs, bs), 0) // slices
        c = jax.lax.broadcasted_iota(jnp.int32, (bs, bs), 1) // slices
        same_graph = r == c
        for h in range(heads):
            a = acc_ref[h]
            tok = a[:, :dh] / (a[:, dh:dh + 1] + 1e-5)
            q = jnp.dot(tok, wq_ref[...], preferred_element_type=jnp.float32)
            k = jnp.dot(tok, wk_ref[...], preferred_element_type=jnp.float32)
            v = jnp.dot(tok, wv_ref[...], preferred_element_type=jnp.float32)
            dots = jax.lax.dot_general(q, k, (((1,), (1,)), ((), ())),
                                       preferred_element_type=jnp.float32)
            dots = jnp.where(same_graph, dots * scale, -1e30)
            mx = jnp.max(dots, axis=-1, keepdims=True)
            e = jnp.exp(dots - mx)
            attn = e / jnp.sum(e, axis=-1, keepdims=True)
            att = jnp.dot(attn, v, preferred_element_type=jnp.float32)
            att2_ref[h] = jnp.dot(att, wout_ref[h * dh:(h + 1) * dh, :],
                                  preferred_element_type=jnp.float32)

    @pl.when(p == 1)
    def _readback():
        sw = sw_ref[pl.ds(i * nb, nb), :]
        onehot = onehot512(bcol_ref[...])
        o = jnp.zeros(out_ref.shape, jnp.float32) + bout_ref[...]
        for h in range(heads):
            sw_h = sw[:, h * slices:(h + 1) * slices]
            comb = onehot * jnp.concatenate([sw_h] * _B, axis=1)
            o = o + jnp.dot(comb, att2_ref[h], preferred_element_type=jnp.float32)
        out_ref[...] = o


def kernel(x, batch, W_fx, b_fx, W_x, b_x, W_slice, b_slice, Wq, Wk, Wv,
           W_out, b_out, g_temp):
    n, dim = x.shape
    heads = g_temp.shape[1]
    inner = W_fx.shape[0]
    dh = inner // heads
    slices = W_slice.shape[0]
    bs = _B * slices

    nb = 2000
    grid = (2, n // nb)

    bcol = batch.astype(jnp.float32).reshape(n, 1)
    eye_h = jnp.eye(heads, dtype=jnp.float32)
    wsl_t = jnp.einsum('hk,sd->hdks', eye_h, W_slice).reshape(inner, heads * slices)
    bsl = jnp.tile(b_slice, heads).reshape(1, heads * slices)
    invg = jnp.repeat(1.0 / g_temp.reshape(heads), slices).reshape(1, heads * slices)
    gs_i = jnp.arange(heads * slices) // slices
    gsum = (gs_i[:, None] == gs_i[None, :]).astype(jnp.float32)

    blk = lambda r, c: pl.BlockSpec((r, c), lambda p, i: (i, 0))
    full = lambda r, c: pl.BlockSpec((r, c), lambda p, i: (0, 0))

    out = pl.pallas_call(
        functools.partial(_fused_kernel, heads, slices, dh, nb),
        grid=grid,
        in_specs=[
            blk(nb, dim),
            blk(nb, 1),
            full(dim, inner), full(1, inner), full(dim, inner), full(1, inner),
            full(inner, heads * slices), full(1, heads * slices),
            full(1, heads * slices), full(heads * slices, heads * slices),
            full(dh, dh), full(dh, dh), full(dh, dh),
            full(inner, dim), full(1, dim),
        ],
        out_specs=blk(nb, dim),
        out_shape=jax.ShapeDtypeStruct((n, dim), jnp.float32),
        scratch_shapes=[
            pltpu.VMEM((n, heads * slices), jnp.float32),
            pltpu.VMEM((heads, bs, dh + 1), jnp.float32),
            pltpu.VMEM((heads, bs, dim), jnp.float32),
        ],
    )(x, bcol, W_fx.T, b_fx.reshape(1, inner), W_x.T, b_x.reshape(1, inner),
      wsl_t, bsl, invg, gsum, Wq.T, Wk.T, Wv.T, W_out.T, b_out.reshape(1, dim))
    return out


# speedup vs baseline: 39.4785x; 1.1256x over previous
import functools

import jax
import jax.numpy as jnp
from jax.experimental import pallas as pl
from jax.experimental.pallas import tpu as pltpu

_B = 16


def _fused_kernel(heads, slices, dh, nb, xb_ref, bcol_ref, wfx_ref, bfx_ref,
                  wx_ref, bx_ref, wsl_ref, bsl_ref, invg_ref, gsum_ref,
                  wq_ref, wk_ref, wv_ref, wout_ref, bout_ref,
                  out_ref, sw_ref, acc_ref, att2_ref):
    p = pl.program_id(0)
    i = pl.program_id(1)
    bs = _B * slices
    scale = float(dh) ** -0.5

    def onehot512(bcol):
        bid = jax.lax.broadcasted_iota(jnp.int32, (1, bs), 1) // slices
        return (bcol == bid.astype(jnp.float32)).astype(jnp.bfloat16)

    @pl.when(p == 0)
    def _pool():
        @pl.when(i == 0)
        def _init():
            acc_ref[...] = jnp.zeros_like(acc_ref)

        xb = xb_ref[...]
        fx = jnp.dot(xb, wfx_ref[...], preferred_element_type=jnp.float32) + bfx_ref[...]
        xm = jnp.dot(xb, wx_ref[...], preferred_element_type=jnp.float32) + bx_ref[...]
        logits = (jnp.dot(xm, wsl_ref[...], preferred_element_type=jnp.float32)
                  + bsl_ref[...]) * invg_ref[...]
        m = jnp.max(logits, axis=-1, keepdims=True)
        e = jnp.exp(logits - m)
        denom = jnp.dot(e, gsum_ref[...], preferred_element_type=jnp.float32)
        sw = (e / denom).astype(jnp.bfloat16)
        sw_ref[pl.ds(i * nb, nb), :] = sw
        fxb = fx.astype(jnp.bfloat16)
        onehot = onehot512(bcol_ref[...])
        ones = jnp.ones((nb, 1), jnp.bfloat16)
        for h in range(heads):
            sw_h = sw[:, h * slices:(h + 1) * slices]
            comb = onehot * jnp.concatenate([sw_h] * _B, axis=1)
            fxa = jnp.concatenate([fxb[:, h * slices:(h + 1) * slices], ones],
                                  axis=1)
            res = jax.lax.dot_general(comb, fxa, (((0,), (0,)), ((), ())),
                                      preferred_element_type=jnp.float32)
            acc_ref[h] = acc_ref[h] + res

    @pl.when((p == 1) & (i == 0))
    def _attend():
        r = jax.lax.broadcasted_iota(jnp.int32, (bs, bs), 0) // slices
        c = jax.lax.broadcasted_iota(jnp.int32, (bs, bs), 1) // slices
        same_graph = r == c
        for h in range(heads):
            a = acc_ref[h]
            tok = a[:, :dh] / (a[:, dh:dh + 1] + 1e-5)
            q = jnp.dot(tok, wq_ref[...], preferred_element_type=jnp.float32)
            k = jnp.dot(tok, wk_ref[...], preferred_element_type=jnp.float32)
            v = jnp.dot(tok, wv_ref[...], preferred_element_type=jnp.float32)
            dots = jax.lax.dot_general(q, k, (((1,), (1,)), ((), ())),
                                       preferred_element_type=jnp.float32)
            dots = jnp.where(same_graph, dots * scale, -1e30)
            mx = jnp.max(dots, axis=-1, keepdims=True)
            e = jnp.exp(dots - mx)
            attn = e / jnp.sum(e, axis=-1, keepdims=True)
            att = jnp.dot(attn, v, preferred_element_type=jnp.float32)
            att2_ref[h] = jnp.dot(att, wout_ref[h * dh:(h + 1) * dh, :],
                                  preferred_element_type=jnp.float32
                                  ).astype(jnp.bfloat16)

    @pl.when(p == 1)
    def _readback():
        sw = sw_ref[pl.ds(i * nb, nb), :]
        onehot = onehot512(bcol_ref[...])
        o = jnp.zeros(out_ref.shape, jnp.float32) + bout_ref[...]
        for h in range(heads):
            sw_h = sw[:, h * slices:(h + 1) * slices]
            comb = onehot * jnp.concatenate([sw_h] * _B, axis=1)
            o = o + jnp.dot(comb, att2_ref[h], preferred_element_type=jnp.float32)
        out_ref[...] = o


def kernel(x, batch, W_fx, b_fx, W_x, b_x, W_slice, b_slice, Wq, Wk, Wv,
           W_out, b_out, g_temp):
    n, dim = x.shape
    heads = g_temp.shape[1]
    inner = W_fx.shape[0]
    dh = inner // heads
    slices = W_slice.shape[0]
    bs = _B * slices

    nb = 2000
    grid = (2, n // nb)

    bcol = batch.astype(jnp.float32).reshape(n, 1)
    eye_h = jnp.eye(heads, dtype=jnp.float32)
    wsl_t = jnp.einsum('hk,sd->hdks', eye_h, W_slice).reshape(inner, heads * slices)
    bsl = jnp.tile(b_slice, heads).reshape(1, heads * slices)
    invg = jnp.repeat(1.0 / g_temp.reshape(heads), slices).reshape(1, heads * slices)
    gs_i = jnp.arange(heads * slices) // slices
    gsum = (gs_i[:, None] == gs_i[None, :]).astype(jnp.float32)

    blk = lambda r, c: pl.BlockSpec((r, c), lambda p, i: (i, 0))
    full = lambda r, c: pl.BlockSpec((r, c), lambda p, i: (0, 0))

    out = pl.pallas_call(
        functools.partial(_fused_kernel, heads, slices, dh, nb),
        grid=grid,
        in_specs=[
            blk(nb, dim),
            blk(nb, 1),
            full(dim, inner), full(1, inner), full(dim, inner), full(1, inner),
            full(inner, heads * slices), full(1, heads * slices),
            full(1, heads * slices), full(heads * slices, heads * slices),
            full(dh, dh), full(dh, dh), full(dh, dh),
            full(inner, dim), full(1, dim),
        ],
        out_specs=blk(nb, dim),
        out_shape=jax.ShapeDtypeStruct((n, dim), jnp.float32),
        scratch_shapes=[
            pltpu.VMEM((n, heads * slices), jnp.bfloat16),
            pltpu.VMEM((heads, bs, dh + 1), jnp.float32),
            pltpu.VMEM((heads, bs, dim), jnp.bfloat16),
        ],
    )(x, bcol, W_fx.T, b_fx.reshape(1, inner), W_x.T, b_x.reshape(1, inner),
      wsl_t, bsl, invg, gsum, Wq.T, Wk.T, Wv.T, W_out.T, b_out.reshape(1, dim))
    return out


# speedup vs baseline: 47.6821x; 1.2078x over previous
import functools

import jax
import jax.numpy as jnp
from jax.experimental import pallas as pl
from jax.experimental.pallas import tpu as pltpu

_B = 16


def _fused_kernel(heads, slices, dh, nb, xb_ref, bcol_ref, wfx_ref, bfx_ref,
                  wx_ref, bx_ref, wsl_ref, bsl_ref, invg_ref, gsum_ref,
                  wq_ref, wk_ref, wv_ref, wout_ref, bout_ref,
                  out_ref, sw_ref, acc_ref, att2_ref):
    p = pl.program_id(0)
    i = pl.program_id(1)
    bs = _B * slices
    scale = float(dh) ** -0.5

    def onehot512(bcol):
        bid = jax.lax.broadcasted_iota(jnp.int32, (1, bs), 1) // slices
        return (bcol == bid.astype(jnp.float32)).astype(jnp.bfloat16)

    @pl.when(p == 0)
    def _pool():
        @pl.when(i == 0)
        def _init():
            acc_ref[...] = jnp.zeros_like(acc_ref)

        xb = xb_ref[...]
        fx = jnp.dot(xb, wfx_ref[...], preferred_element_type=jnp.float32) + bfx_ref[...]
        xm = jnp.dot(xb, wx_ref[...], preferred_element_type=jnp.float32) + bx_ref[...]
        logits = (jnp.dot(xm, wsl_ref[...], preferred_element_type=jnp.float32)
                  + bsl_ref[...]) * invg_ref[...]
        m = jnp.max(logits, axis=-1, keepdims=True)
        e = jnp.exp(logits - m)
        denom = jnp.dot(e, gsum_ref[...], preferred_element_type=jnp.float32)
        sw = (e / denom).astype(jnp.bfloat16)
        sw_ref[pl.ds(i * nb, nb), :] = sw
        fxb = fx.astype(jnp.bfloat16)
        onehot = onehot512(bcol_ref[...])
        ones = jnp.ones((nb, 1), jnp.bfloat16)
        for h in range(heads):
            sw_h = sw[:, h * slices:(h + 1) * slices]
            comb = onehot * jnp.concatenate([sw_h] * _B, axis=1)
            fxa = jnp.concatenate([fxb[:, h * slices:(h + 1) * slices], ones],
                                  axis=1)
            res = jax.lax.dot_general(fxa, comb, (((0,), (0,)), ((), ())),
                                      preferred_element_type=jnp.float32)
            acc_ref[h] = acc_ref[h] + res

    @pl.when((p == 1) & (i == 0))
    def _attend():
        r = jax.lax.broadcasted_iota(jnp.int32, (bs, bs), 0) // slices
        c = jax.lax.broadcasted_iota(jnp.int32, (bs, bs), 1) // slices
        same_graph = r == c
        for h in range(heads):
            a = acc_ref[h]
            tok_t = a[:dh, :] / (a[dh:dh + 1, :] + 1e-5)
            q_t = jnp.dot(wq_ref[...], tok_t, preferred_element_type=jnp.float32)
            k_t = jnp.dot(wk_ref[...], tok_t, preferred_element_type=jnp.float32)
            v_t = jnp.dot(wv_ref[...], tok_t, preferred_element_type=jnp.float32)
            dots = jax.lax.dot_general(q_t, k_t, (((0,), (0,)), ((), ())),
                                       preferred_element_type=jnp.float32)
            dots = jnp.where(same_graph, dots * scale, -1e30)
            mx = jnp.max(dots, axis=-1, keepdims=True)
            e = jnp.exp(dots - mx)
            attn = e / jnp.sum(e, axis=-1, keepdims=True)
            att_t = jax.lax.dot_general(v_t, attn, (((1,), (1,)), ((), ())),
                                        preferred_element_type=jnp.float32)
            att2_ref[h] = jax.lax.dot_general(
                att_t, wout_ref[h * dh:(h + 1) * dh, :],
                (((0,), (0,)), ((), ())),
                preferred_element_type=jnp.float32).astype(jnp.bfloat16)

    @pl.when(p == 1)
    def _readback():
        sw = sw_ref[pl.ds(i * nb, nb), :]
        onehot = onehot512(bcol_ref[...])
        o = jnp.zeros(out_ref.shape, jnp.float32) + bout_ref[...]
        for h in range(heads):
            sw_h = sw[:, h * slices:(h + 1) * slices]
            comb = onehot * jnp.concatenate([sw_h] * _B, axis=1)
            o = o + jnp.dot(comb, att2_ref[h], preferred_element_type=jnp.float32)
        out_ref[...] = o


def kernel(x, batch, W_fx, b_fx, W_x, b_x, W_slice, b_slice, Wq, Wk, Wv,
           W_out, b_out, g_temp):
    n, dim = x.shape
    heads = g_temp.shape[1]
    inner = W_fx.shape[0]
    dh = inner // heads
    slices = W_slice.shape[0]
    bs = _B * slices

    nb = 2000
    grid = (2, n // nb)

    bcol = batch.astype(jnp.float32).reshape(n, 1)
    eye_h = jnp.eye(heads, dtype=jnp.float32)
    wsl_t = jnp.einsum('hk,sd->hdks', eye_h, W_slice).reshape(inner, heads * slices)
    bsl = jnp.tile(b_slice, heads).reshape(1, heads * slices)
    invg = jnp.repeat(1.0 / g_temp.reshape(heads), slices).reshape(1, heads * slices)
    gs_i = jnp.arange(heads * slices) // slices
    gsum = (gs_i[:, None] == gs_i[None, :]).astype(jnp.float32)

    blk = lambda r, c: pl.BlockSpec((r, c), lambda p, i: (i, 0))
    full = lambda r, c: pl.BlockSpec((r, c), lambda p, i: (0, 0))

    out = pl.pallas_call(
        functools.partial(_fused_kernel, heads, slices, dh, nb),
        grid=grid,
        in_specs=[
            blk(nb, dim),
            blk(nb, 1),
            full(dim, inner), full(1, inner), full(dim, inner), full(1, inner),
            full(inner, heads * slices), full(1, heads * slices),
            full(1, heads * slices), full(heads * slices, heads * slices),
            full(dh, dh), full(dh, dh), full(dh, dh),
            full(inner, dim), full(1, dim),
        ],
        out_specs=blk(nb, dim),
        out_shape=jax.ShapeDtypeStruct((n, dim), jnp.float32),
        scratch_shapes=[
            pltpu.VMEM((n, heads * slices), jnp.bfloat16),
            pltpu.VMEM((heads, dh + 1, bs), jnp.float32),
            pltpu.VMEM((heads, bs, dim), jnp.bfloat16),
        ],
    )(x, bcol, W_fx.T, b_fx.reshape(1, inner), W_x.T, b_x.reshape(1, inner),
      wsl_t, bsl, invg, gsum, Wq, Wk, Wv, W_out.T, b_out.reshape(1, dim))
    return out
